# baseline (speedup 1.0000x reference)
import jax
import jax.numpy as jnp
import numpy as np
from jax.experimental import pallas as pl
from jax.experimental.pallas import tpu as pltpu

B_ORT, B_REG, B_CLS = 0.2, 2.0, 1.0
_B = 4
_P = 40000
_PP = 40960
_RR = 320
_CLS_TOTAL = float(_B * 2 * _P)


def _build_selectors():
    s_rt = np.zeros((2304, 2304), np.float32)
    for q in range(256):
        for c in range(9):
            s_rt[9 * q + c, 128 * (9 * (q % 2) + c) + q // 2] = 1.0
    s_t = np.zeros((256, 256), np.float32)
    for pp in range(128):
        for c in range(2):
            s_t[2 * pp + c, 128 * c + pp] = 1.0
    return (jnp.asarray(s_rt, dtype=jnp.bfloat16),
            jnp.asarray(s_t, dtype=jnp.bfloat16))


def _loss_kernel(x_ref, t_ref, rg_ref, rt_ref, srt_ref, st_ref,
                 out_ref, acc_ref):
    b = pl.program_id(0)

    @pl.when(b == 0)
    def _init():
        for i in range(4):
            acc_ref[i] = 0.0

    d_rt = jax.lax.dot_general(
        rt_ref[0].astype(jnp.bfloat16), srt_ref[...],
        (((1,), (0,)), ((), ())), preferred_element_type=jnp.float32)
    d_t = jax.lax.dot_general(
        t_ref[0].astype(jnp.bfloat16), st_ref[...],
        (((1,), (0,)), ((), ())), preferred_element_type=jnp.float32)

    def slab(arr, k):
        return jax.lax.slice(arr, (0, 128 * k), (_RR, 128 * (k + 1)))

    pos = (jax.lax.broadcasted_iota(jnp.int32, (_RR, 128), 0) * 128
           + jax.lax.broadcasted_iota(jnp.int32, (_RR, 128), 1))
    valid = (pos < _P).astype(jnp.float32)

    cls_sum = 0.0
    for c in range(2):
        x = x_ref[0, c]
        t = slab(d_t, c)
        p = jax.nn.sigmoid(x)
        pt = jnp.where(t == 1.0, p, 1.0 - p)
        at = jnp.where(t == 1.0, 1000.0, 1.0)
        qf = 1.0 - pt
        w = at * qf * qf * valid
        bce = jnp.maximum(x, 0.0) - x * t + jnp.log1p(jnp.exp(-jnp.abs(x)))
        cls_sum += jnp.sum(w * bce)

    sl1_sum = 0.0
    ce_sum = 0.0
    npos = 0.0
    for a in range(2):
        mask = (slab(d_rt, 9 * a) == 1.0).astype(jnp.float32)
        npos += jnp.sum(mask)
        for j in range(7):
            s = rg_ref[0, 9 * a + j]
            if a == 0 and j == 6:
                s = jnp.tanh(s)
            d = s - slab(d_rt, 9 * a + j + 1)
            ad = jnp.abs(d)
            sl1 = jnp.where(ad < 1.0, 0.5 * d * d, ad - 0.5)
            sl1_sum += jnp.sum(sl1 * mask)
        z0 = rg_ref[0, 9 * a + 7]
        z1 = rg_ref[0, 9 * a + 8]
        tc = slab(d_rt, 9 * a + 8)
        diff = jnp.where(tc == 1.0, z0 - z1, z1 - z0)
        ce = jnp.maximum(diff, 0.0) + jnp.log1p(jnp.exp(-jnp.abs(diff)))
        ce_sum += jnp.sum(ce * mask)

    acc_ref[0] += cls_sum
    acc_ref[1] += sl1_sum
    acc_ref[2] += ce_sum
    acc_ref[3] += npos

    @pl.when(b == _B - 1)
    def _final():
        n_pos = acc_ref[3]
        cls_loss = acc_ref[0] / _CLS_TOTAL
        reg_loss = acc_ref[1] / (n_pos * 7.0)
        ort_loss = acc_ref[2] / n_pos
        loss = B_CLS * cls_loss + B_ORT * ort_loss + B_REG * reg_loss
        out_ref[...] = jnp.full((1, 1), loss, dtype=jnp.float32)


def _pad_rows(flat, row):
    n = flat.shape[-1]
    return jnp.pad(flat, ((0, 0), (0, _RR * row - n))).reshape(
        flat.shape[0], _RR, row)


def kernel(cls_tensor, reg_tensor, cls_targets, reg_targets):
    x = jnp.pad(cls_tensor.reshape(_B, 2, _P),
                ((0, 0), (0, 0), (0, _PP - _P))).reshape(_B, 2, _RR, 128)
    t = _pad_rows(cls_targets.reshape(_B, 2 * _P), 256)
    rg = jnp.pad(reg_tensor.reshape(_B, 18, _P),
                 ((0, 0), (0, 0), (0, _PP - _P))).reshape(_B, 18, _RR, 128)
    rt = _pad_rows(reg_targets.reshape(_B, 9 * 2 * _P), 2304)
    s_rt, s_t = _build_selectors()

    out = pl.pallas_call(
        _loss_kernel,
        grid=(_B,),
        in_specs=[
            pl.BlockSpec((1, 2, _RR, 128), lambda b: (b, 0, 0, 0)),
            pl.BlockSpec((1, _RR, 256), lambda b: (b, 0, 0)),
            pl.BlockSpec((1, 18, _RR, 128), lambda b: (b, 0, 0, 0)),
            pl.BlockSpec((1, _RR, 2304), lambda b: (b, 0, 0)),
            pl.BlockSpec((2304, 2304), lambda b: (0, 0)),
            pl.BlockSpec((256, 256), lambda b: (0, 0)),
        ],
        out_specs=pl.BlockSpec((1, 1), lambda b: (0, 0)),
        out_shape=jax.ShapeDtypeStruct((1, 1), jnp.float32),
        scratch_shapes=[pltpu.SMEM((4,), jnp.float32)],
    )(x, t, rg, rt, s_rt, s_t)
    return out[0, 0]


# speedup vs baseline: 3.0617x; 3.0617x over previous
import jax
import jax.numpy as jnp
from jax.experimental import pallas as pl
from jax.experimental.pallas import tpu as pltpu

B_ORT, B_REG, B_CLS = 0.2, 2.0, 1.0
_B = 4
_P = 40000
_SUB, _LANE = 8, 5000
_CLS_TOTAL = float(_B * 2 * _P)


def _loss_kernel(x_ref, t_ref, rg_ref, rt_ref, out_ref, acc_ref):
    b = pl.program_id(0)

    @pl.when(b == 0)
    def _init():
        for i in range(4):
            acc_ref[i] = 0.0

    x = x_ref[0]
    t = t_ref[0]
    p = jax.nn.sigmoid(x)
    pt = jnp.where(t == 1.0, p, 1.0 - p)
    at = jnp.where(t == 1.0, 1000.0, 1.0)
    q = 1.0 - pt
    w = at * q * q
    bce = jnp.maximum(x, 0.0) - x * t + jnp.log1p(jnp.exp(-jnp.abs(x)))
    cls_sum = jnp.sum(w * bce)

    sl1_sum = 0.0
    ce_sum = 0.0
    npos = 0.0
    rows7 = jax.lax.broadcasted_iota(jnp.int32, (7, _SUB, _LANE), 0)
    for a in range(2):
        mask = (rt_ref[0, 9 * a] == 1.0).astype(jnp.float32)
        npos += jnp.sum(mask)
        s = rg_ref[0, 9 * a:9 * a + 7]
        if a == 0:
            s = jnp.where(rows7 == 6, jnp.tanh(s), s)
        d = s - rt_ref[0, 9 * a + 1:9 * a + 8]
        ad = jnp.abs(d)
        sl1 = jnp.where(ad < 1.0, 0.5 * d * d, ad - 0.5)
        sl1_sum += jnp.sum(sl1 * mask[None])
        z0 = rg_ref[0, 9 * a + 7]
        z1 = rg_ref[0, 9 * a + 8]
        tc = rt_ref[0, 9 * a + 8]
        diff = jnp.where(tc == 1.0, z0 - z1, z1 - z0)
        ce = jnp.maximum(diff, 0.0) + jnp.log1p(jnp.exp(-jnp.abs(diff)))
        ce_sum += jnp.sum(ce * mask)

    acc_ref[0] += cls_sum
    acc_ref[1] += sl1_sum
    acc_ref[2] += ce_sum
    acc_ref[3] += npos

    @pl.when(b == _B - 1)
    def _final():
        n_pos = acc_ref[3]
        cls_loss = acc_ref[0] / _CLS_TOTAL
        reg_loss = acc_ref[1] / (n_pos * 7.0)
        ort_loss = acc_ref[2] / n_pos
        loss = B_CLS * cls_loss + B_ORT * ort_loss + B_REG * reg_loss
        out_ref[...] = jnp.full((1, 1), loss, dtype=jnp.float32)


def kernel(cls_tensor, reg_tensor, cls_targets, reg_targets):
    x = cls_tensor.reshape(_B, 2, _SUB, _LANE)
    t = cls_targets.transpose(0, 3, 1, 2).reshape(_B, 2, _SUB, _LANE)
    rg = reg_tensor.reshape(_B, 18, _SUB, _LANE)
    rt = (reg_targets.reshape(_B, _P, 2, 9)
          .transpose(0, 2, 3, 1)
          .reshape(_B, 18, _SUB, _LANE))

    out = pl.pallas_call(
        _loss_kernel,
        grid=(_B,),
        in_specs=[
            pl.BlockSpec((1, 2, _SUB, _LANE), lambda b: (b, 0, 0, 0)),
            pl.BlockSpec((1, 2, _SUB, _LANE), lambda b: (b, 0, 0, 0)),
            pl.BlockSpec((1, 18, _SUB, _LANE), lambda b: (b, 0, 0, 0)),
            pl.BlockSpec((1, 18, _SUB, _LANE), lambda b: (b, 0, 0, 0)),
        ],
        out_specs=pl.BlockSpec((1, 1), lambda b: (0, 0)),
        out_shape=jax.ShapeDtypeStruct((1, 1), jnp.float32),
        scratch_shapes=[pltpu.SMEM((4,), jnp.float32)],
    )(x, t, rg, rt)
    return out[0, 0]


# speedup vs baseline: 3.1889x; 1.0415x over previous
import jax
import jax.numpy as jnp
from jax.experimental import pallas as pl
from jax.experimental.pallas import tpu as pltpu

B_ORT, B_REG, B_CLS = 0.2, 2.0, 1.0
_B = 4
_P = 40000
_SUB, _LANE = 16, 2500
_CLS_TOTAL = float(_B * 2 * _P)


def _loss_kernel(x_ref, t_ref, rg_ref, rt_ref, out_ref, acc_ref):
    b = pl.program_id(0)

    @pl.when(b == 0)
    def _init():
        for i in range(4):
            acc_ref[i] = 0.0

    x = x_ref[0]
    t = t_ref[0].astype(jnp.float32)
    p = jax.nn.sigmoid(x)
    pt = jnp.where(t == 1.0, p, 1.0 - p)
    at = jnp.where(t == 1.0, 1000.0, 1.0)
    q = 1.0 - pt
    w = at * q * q
    bce = jnp.maximum(x, 0.0) - x * t + jnp.log1p(jnp.exp(-jnp.abs(x)))
    cls_sum = jnp.sum(w * bce)

    sl1_sum = 0.0
    ce_sum = 0.0
    npos = 0.0
    rows7 = jax.lax.broadcasted_iota(jnp.int32, (7, _SUB, _LANE), 0)
    for a in range(2):
        mask = (rt_ref[0, 9 * a] == 1).astype(jnp.float32)
        npos += jnp.sum(mask)
        s = rg_ref[0, 9 * a:9 * a + 7]
        if a == 0:
            s = jnp.where(rows7 == 6, jnp.tanh(s), s)
        d = s - rt_ref[0, 9 * a + 1:9 * a + 8].astype(jnp.float32)
        ad = jnp.abs(d)
        sl1 = jnp.where(ad < 1.0, 0.5 * d * d, ad - 0.5)
        sl1_sum += jnp.sum(sl1 * mask[None])
        z0 = rg_ref[0, 9 * a + 7]
        z1 = rg_ref[0, 9 * a + 8]
        tc = rt_ref[0, 9 * a + 8]
        diff = jnp.where(tc == 1, z0 - z1, z1 - z0)
        ce = jnp.maximum(diff, 0.0) + jnp.log1p(jnp.exp(-jnp.abs(diff)))
        ce_sum += jnp.sum(ce * mask)

    acc_ref[0] += cls_sum
    acc_ref[1] += sl1_sum
    acc_ref[2] += ce_sum
    acc_ref[3] += npos

    @pl.when(b == _B - 1)
    def _final():
        n_pos = acc_ref[3]
        cls_loss = acc_ref[0] / _CLS_TOTAL
        reg_loss = acc_ref[1] / (n_pos * 7.0)
        ort_loss = acc_ref[2] / n_pos
        loss = B_CLS * cls_loss + B_ORT * ort_loss + B_REG * reg_loss
        out_ref[...] = jnp.full((1, 1), loss, dtype=jnp.float32)


def kernel(cls_tensor, reg_tensor, cls_targets, reg_targets):
    x = cls_tensor.reshape(_B, 2, _SUB, _LANE)
    t = (cls_targets.astype(jnp.bfloat16)
         .transpose(0, 3, 1, 2).reshape(_B, 2, _SUB, _LANE))
    rg = reg_tensor.reshape(_B, 18, _SUB, _LANE)
    rt = (reg_targets.astype(jnp.bfloat16).reshape(_B, _P, 2, 9)
          .transpose(0, 2, 3, 1)
          .reshape(_B, 18, _SUB, _LANE))

    out = pl.pallas_call(
        _loss_kernel,
        grid=(_B,),
        in_specs=[
            pl.BlockSpec((1, 2, _SUB, _LANE), lambda b: (b, 0, 0, 0)),
            pl.BlockSpec((1, 2, _SUB, _LANE), lambda b: (b, 0, 0, 0)),
            pl.BlockSpec((1, 18, _SUB, _LANE), lambda b: (b, 0, 0, 0)),
            pl.BlockSpec((1, 18, _SUB, _LANE), lambda b: (b, 0, 0, 0)),
        ],
        out_specs=pl.BlockSpec((1, 1), lambda b: (0, 0)),
        out_shape=jax.ShapeDtypeStruct((1, 1), jnp.float32),
        scratch_shapes=[pltpu.SMEM((4,), jnp.float32)],
    )(x, t, rg, rt)
    return out[0, 0]
